# baseline (speedup 1.0000x reference)
import functools

import jax
import jax.numpy as jnp
from jax import lax
from jax.experimental import pallas as pl
from jax.experimental.pallas import tpu as pltpu
from jax.experimental.pallas import tpu_sc as plsc

_NUM_CORES = 2
_NUM_SUBCORES = 16
_LANES = 16
_NW = _NUM_CORES * _NUM_SUBCORES


@functools.lru_cache(maxsize=None)
def _make_sc_kernel(batch: int, n_sites: int):
    b_per_w = batch // _NW
    assert batch % (8 * _NW) == 0
    mesh = plsc.VectorSubcoreMesh(
        core_axis_name="c", subcore_axis_name="s")

    @functools.partial(
        pl.kernel,
        out_type=(
            jax.ShapeDtypeStruct((batch,), jnp.float32),
            jax.ShapeDtypeStruct((batch,), jnp.float32),
        ),
        mesh=mesh,
        compiler_params=pltpu.CompilerParams(
            needs_layout_passes=False, use_tc_tiling_on_sc=False),
        scratch_types=[
            pltpu.VMEM((b_per_w, n_sites), jnp.int32),
            pltpu.VMEM((b_per_w,), jnp.int32),
            pltpu.VMEM((b_per_w,), jnp.float32),
            pltpu.VMEM((b_per_w,), jnp.float32),
            pltpu.SemaphoreType.DMA,
        ],
    )
    def sc_kernel(x_hbm, real_hbm, imag_hbm, out_r, out_i,
                  xv, idxv, rv, iv, sem):
        wid = lax.axis_index("s") * _NUM_CORES + lax.axis_index("c")
        base = wid * b_per_w
        pltpu.sync_copy(x_hbm.at[pl.ds(base, b_per_w), :], xv)

        lanes = lax.iota(jnp.int32, _LANES)

        def body(i, carry):
            rows = i * _LANES + lanes
            acc = jnp.zeros((_LANES,), jnp.int32)
            for j in range(n_sites):
                cols = jnp.full((_LANES,), j, jnp.int32)
                xj = plsc.load_gather(xv, [rows, cols])
                acc = acc * 2 + ((1 - xj) >> 1)
            off = pl.multiple_of(i * _LANES, _LANES)
            idxv[pl.ds(off, _LANES)] = acc
            return carry

        lax.fori_loop(0, b_per_w // _LANES, body, 0)

        pltpu.async_copy(real_hbm.at[idxv], rv, sem).wait()
        pltpu.async_copy(imag_hbm.at[idxv], iv, sem).wait()
        pltpu.sync_copy(rv, out_r.at[pl.ds(base, b_per_w)])
        pltpu.sync_copy(iv, out_i.at[pl.ds(base, b_per_w)])

    return sc_kernel


def kernel(x, real, imag):
    batch, n_sites = x.shape
    r, i = _make_sc_kernel(batch, n_sites)(x, real, imag)
    return lax.complex(r, i)


# speedup vs baseline: 1.6604x; 1.6604x over previous
import functools

import jax
import jax.numpy as jnp
from jax import lax
from jax.experimental import pallas as pl
from jax.experimental.pallas import tpu as pltpu
from jax.experimental.pallas import tpu_sc as plsc

_NUM_CORES = 2
_NUM_SUBCORES = 16
_LANES = 16
_NW = _NUM_CORES * _NUM_SUBCORES


@functools.lru_cache(maxsize=None)
def _make_sc_kernel(batch: int, n_sites: int):
    b_per_w = batch // _NW
    assert batch % (8 * _NW) == 0
    mesh = plsc.VectorSubcoreMesh(
        core_axis_name="c", subcore_axis_name="s")

    @functools.partial(
        pl.kernel,
        out_type=(
            jax.ShapeDtypeStruct((batch,), jnp.float32),
            jax.ShapeDtypeStruct((batch,), jnp.float32),
        ),
        mesh=mesh,
        compiler_params=pltpu.CompilerParams(needs_layout_passes=False),
        scratch_types=[
            pltpu.VMEM((n_sites, b_per_w), jnp.int32),
            pltpu.VMEM((b_per_w,), jnp.int32),
            pltpu.VMEM((b_per_w,), jnp.float32),
            pltpu.VMEM((b_per_w,), jnp.float32),
            pltpu.SemaphoreType.DMA,
        ],
    )
    def sc_kernel(xt_hbm, real_hbm, imag_hbm, out_r, out_i,
                  xtv, idxv, rv, iv, sem):
        wid = lax.axis_index("s") * _NUM_CORES + lax.axis_index("c")
        base = wid * b_per_w
        pltpu.sync_copy(xt_hbm.at[:, pl.ds(base, b_per_w)], xtv)

        def body(i, carry):
            off = pl.multiple_of(i * _LANES, _LANES)
            acc = jnp.zeros((_LANES,), jnp.int32)
            for j in range(n_sites):
                v = xtv[j, pl.ds(off, _LANES)]
                acc = acc * 2 + ((1 - v) >> 1)
            idxv[pl.ds(off, _LANES)] = acc
            return carry

        lax.fori_loop(0, b_per_w // _LANES, body, 0)

        pltpu.async_copy(real_hbm.at[idxv], rv, sem).wait()
        pltpu.async_copy(imag_hbm.at[idxv], iv, sem).wait()
        pltpu.sync_copy(rv, out_r.at[pl.ds(base, b_per_w)])
        pltpu.sync_copy(iv, out_i.at[pl.ds(base, b_per_w)])

    return sc_kernel


def kernel(x, real, imag):
    batch, n_sites = x.shape
    r, i = _make_sc_kernel(batch, n_sites)(x.T, real, imag)
    return lax.complex(r, i)


# speedup vs baseline: 1.6963x; 1.0216x over previous
import functools

import jax
import jax.numpy as jnp
from jax import lax
from jax.experimental import pallas as pl
from jax.experimental.pallas import tpu as pltpu
from jax.experimental.pallas import tpu_sc as plsc

_NUM_CORES = 2
_NUM_SUBCORES = 16
_LANES = 16
_NW = _NUM_CORES * _NUM_SUBCORES


@functools.lru_cache(maxsize=None)
def _make_sc_kernel(batch: int, n_sites: int):
    b_per_w = batch // _NW
    assert batch % (8 * _NW) == 0
    mesh = plsc.VectorSubcoreMesh(
        core_axis_name="c", subcore_axis_name="s")

    @functools.partial(
        pl.kernel,
        out_type=(
            jax.ShapeDtypeStruct((batch,), jnp.float32),
            jax.ShapeDtypeStruct((batch,), jnp.float32),
        ),
        mesh=mesh,
        compiler_params=pltpu.CompilerParams(needs_layout_passes=False),
        scratch_types=[
            pltpu.VMEM((n_sites, b_per_w), jnp.int32),
            pltpu.VMEM((b_per_w,), jnp.int32),
            pltpu.VMEM((b_per_w,), jnp.float32),
            pltpu.VMEM((b_per_w,), jnp.float32),
            pltpu.SemaphoreType.DMA,
            pltpu.SemaphoreType.DMA,
            pltpu.SemaphoreType.DMA,
        ],
    )
    def sc_kernel(xt_hbm, real_hbm, imag_hbm, out_r, out_i,
                  xtv, idxv, rv, iv, sem_a, sem_b, sem_c):
        wid = lax.axis_index("s") * _NUM_CORES + lax.axis_index("c")
        base = wid * b_per_w
        half = b_per_w // 2
        cp_a = pltpu.async_copy(
            xt_hbm.at[:, pl.ds(base, half)],
            xtv.at[:, pl.ds(0, half)], sem_a)
        cp_b = pltpu.async_copy(
            xt_hbm.at[:, pl.ds(base + half, half)],
            xtv.at[:, pl.ds(half, half)], sem_b)

        def body(i, carry):
            off = pl.multiple_of(i * _LANES, _LANES)
            acc = jnp.zeros((_LANES,), jnp.int32)
            for j in range(n_sites):
                v = xtv[j, pl.ds(off, _LANES)]
                acc = acc * 2 + ((1 - v) >> 1)
            idxv[pl.ds(off, _LANES)] = acc
            return carry

        groups = b_per_w // _LANES
        cp_a.wait()
        lax.fori_loop(0, groups // 2, body, 0)
        cp_b.wait()
        lax.fori_loop(groups // 2, groups, body, 0)

        cp_r = pltpu.async_copy(real_hbm.at[idxv], rv, sem_a)
        cp_i = pltpu.async_copy(imag_hbm.at[idxv], iv, sem_b)
        cp_r.wait()
        cp_o = pltpu.async_copy(rv, out_r.at[pl.ds(base, b_per_w)], sem_c)
        cp_i.wait()
        pltpu.sync_copy(iv, out_i.at[pl.ds(base, b_per_w)])
        cp_o.wait()

    return sc_kernel


def kernel(x, real, imag):
    batch, n_sites = x.shape
    r, i = _make_sc_kernel(batch, n_sites)(x.T, real, imag)
    return lax.complex(r, i)
